# baseline (speedup 1.0000x reference)
import numpy as np
import jax
import jax.numpy as jnp
from jax.experimental import pallas as pl
from jax.experimental.pallas import tpu as pltpu

B = 4
NUM_CLASS = 21
TOP_K = 50
CH = 64
W = 7
NOBJ = 20
OTH = 0.3
CTH = 0.01

PREV = NUM_CLASS * TOP_K
N = B * PREV
P = W * W
SLOTS = 64
OS = 56
NB = 24
NH = 1 + NUM_CLASS + 4
HZ = 16
RH = SLOTS * NB
RR = 2 * RH

_POOL = np.zeros((2 * NB, RR), np.float32)
for _n in range(NB):
    for _y in range(W):
        for _x in range(W):
            _r = _n * SLOTS + 8 * _y + _x
            _POOL[_n, _r] = 1.0 / P
            _POOL[NB + _n, RH + _r] = 1.0 / P


def _match_kernel(prev_ref, tgt_ref, aux_ref, nm_ref):
    b = pl.program_id(0)
    p = prev_ref[0]
    t = tgt_ref[0]
    conf = p[:, 0:1]
    px1, py1, px2, py2 = p[:, 1:2], p[:, 2:3], p[:, 3:4], p[:, 4:5]
    tx1, ty1, tx2, ty2, tcl = t[0:1], t[1:2], t[2:3], t[3:4], t[4:5]

    ix = jnp.clip(jnp.minimum(tx2, px2) - jnp.maximum(tx1, px1), 0.0, None)
    iy = jnp.clip(jnp.minimum(ty2, py2) - jnp.maximum(ty1, py1), 0.0, None)
    inter = ix * iy
    area_t = (tx2 - tx1) * (ty2 - ty1)
    area_p = (px2 - px1) * (py2 - py1)
    ov = inter / (area_t + area_p - inter)

    best = jnp.max(ov, axis=1, keepdims=True)
    li = jax.lax.broadcasted_iota(jnp.int32, ov.shape, 1)
    bidx = jnp.min(jnp.where(ov == best, li, NOBJ), axis=1, keepdims=True)
    sel = li == bidx

    def pick(row):
        return jnp.sum(jnp.where(sel, row, 0.0), axis=1, keepdims=True)

    cls_m = pick(tcl)
    lx1, ly1, lx2, ly2 = pick(tx1), pick(ty1), pick(tx2), pick(ty2)
    m = jnp.logical_and(best >= OTH, conf >= CTH).astype(jnp.float32)

    aux_ref[0] = jnp.concatenate(
        [m, cls_m, lx1, ly1, lx2, ly2, p[:, 1:5],
         jnp.zeros((PREV, 6), jnp.float32)], axis=1)

    @pl.when(b == 0)
    def _():
        nm_ref[0, 0] = 0.0

    nm_ref[0, 0] += jnp.sum(m)


def _disc_kernel(ft_ref, fs_ref, aux_ref, w2_ref, bc_ref, wh_ref, bh_ref,
                 pool_ref, acc_ref, xt_ref, xs_ref):
    i = pl.program_id(0)

    @pl.when(i == 0)
    def _():
        xt_ref[0:HZ, :] = jnp.zeros((HZ, CH), jnp.bfloat16)
        xs_ref[0:HZ, :] = jnp.zeros((HZ, CH), jnp.bfloat16)
        xt_ref[HZ + RH:, :] = jnp.zeros((HZ, CH), jnp.bfloat16)
        xs_ref[HZ + RH:, :] = jnp.zeros((HZ, CH), jnp.bfloat16)

    xt_ref[HZ:HZ + RH, :] = \
        jnp.transpose(ft_ref[...], (0, 2, 1)).reshape(RH, CH)
    xs_ref[HZ:HZ + RH, :] = \
        jnp.transpose(fs_ref[...], (0, 2, 1)).reshape(RH, CH)

    cols = []
    for x_ref in (xt_ref, xs_ref):
        Xc = x_ref[...]
        cols.append({0: Xc, 1: jnp.roll(Xc, -1, axis=0),
                     -1: jnp.roll(Xc, 1, axis=0)})
    pieces = []
    for ky in range(3):
        for kx in range(3):
            a = HZ + (ky - 1) * 8
            pieces.append(jnp.concatenate(
                [cols[0][kx - 1][a:a + RH, :],
                 cols[1][kx - 1][a:a + RH, :]], axis=0))
    col = jnp.concatenate(pieces, axis=1)

    h = jnp.dot(col, w2_ref[...], preferred_element_type=jnp.float32)
    h = jnp.maximum(h + bc_ref[...], 0.0)

    hb = jnp.dot(pool_ref[...], h.astype(jnp.bfloat16),
                 preferred_element_type=jnp.float32)

    heads = jnp.dot(hb, wh_ref[...], preferred_element_type=jnp.float32)
    heads = heads + bh_ref[...]
    Ht = heads[:NB]
    Hs = heads[NB:]

    aux = aux_ref[...]
    m = aux[:, 0:1]
    cls = aux[:, 1:2].astype(jnp.int32)
    locm = aux[:, 2:6]
    ploc = aux[:, 6:10]

    pt = jnp.clip(jax.nn.sigmoid(Ht[:, 0:1]), 1e-7, 1.0 - 1e-7)
    ps = jnp.clip(jax.nn.sigmoid(Hs[:, 0:1]), 1e-7, 1.0 - 1e-7)
    real = -jnp.log(pt) * m
    fake = -jnp.log(1.0 - ps) * m

    logits = Hs[:, 1:1 + NUM_CLASS]
    mx = jnp.max(logits, axis=1, keepdims=True)
    lz = jnp.log(jnp.sum(jnp.exp(logits - mx), axis=1, keepdims=True)) + mx
    ci = jax.lax.broadcasted_iota(jnp.int32, logits.shape, 1)
    lsel = jnp.sum(jnp.where(ci == cls, logits, 0.0), axis=1, keepdims=True)
    ce = (lz - lsel) * m

    dl = Ht[:, 1 + NUM_CLASS:NH] + ploc
    diff = dl - locm
    ad = jnp.abs(diff)
    sl1 = jnp.where(ad < 1.0, 0.5 * diff * diff, ad - 0.5)
    locc = jnp.sum(sl1, axis=1, keepdims=True) * m

    part = jnp.concatenate([ce, locc, real, fake], axis=1)
    psum = jnp.sum(part, axis=0, keepdims=True)

    @pl.when(i == 0)
    def _():
        acc_ref[...] = jnp.zeros_like(acc_ref)

    acc_ref[...] += psum


def _pad_cast(f):
    fp = jnp.pad(f.reshape(N, CH, W, W), ((0, 0), (0, 0), (0, 1), (0, 1)))
    return fp.reshape(N, CH, SLOTS).astype(jnp.bfloat16)


def kernel(feature_t, feature_s, prev_t, target, Wc, bc, Wd, bd, Wcl, bcl,
           Wl, bl):
    ft = _pad_cast(feature_t)
    fs = _pad_cast(feature_s)
    prev_flat = prev_t.reshape(B, PREV, 5)
    tgt_T = target.transpose(0, 2, 1)
    W2 = Wc.transpose(2, 3, 1, 0).reshape(9 * CH, CH).astype(jnp.bfloat16)
    Whead = jnp.concatenate([Wd, Wcl, Wl], axis=1)
    bhead = jnp.concatenate([bd, bcl, bl], axis=0)[None, :]
    bc2 = bc[None, :]
    pool = jnp.asarray(_POOL, dtype=jnp.bfloat16)

    aux, nm = pl.pallas_call(
        _match_kernel,
        grid=(B,),
        in_specs=[
            pl.BlockSpec((1, PREV, 5), lambda b: (b, 0, 0)),
            pl.BlockSpec((1, 5, NOBJ), lambda b: (b, 0, 0)),
        ],
        out_specs=[
            pl.BlockSpec((1, PREV, 16), lambda b: (b, 0, 0)),
            pl.BlockSpec(memory_space=pltpu.SMEM),
        ],
        out_shape=[
            jax.ShapeDtypeStruct((B, PREV, 16), jnp.float32),
            jax.ShapeDtypeStruct((1, 1), jnp.float32),
        ],
        compiler_params=pltpu.CompilerParams(
            dimension_semantics=("arbitrary",)),
    )(prev_flat, tgt_T)
    aux_flat = aux.reshape(N, 16)

    acc = pl.pallas_call(
        _disc_kernel,
        grid=(N // NB,),
        in_specs=[
            pl.BlockSpec((NB, CH, SLOTS), lambda i: (i, 0, 0)),
            pl.BlockSpec((NB, CH, SLOTS), lambda i: (i, 0, 0)),
            pl.BlockSpec((NB, 16), lambda i: (i, 0)),
            pl.BlockSpec((9 * CH, CH), lambda i: (0, 0)),
            pl.BlockSpec((1, CH), lambda i: (0, 0)),
            pl.BlockSpec((CH, NH), lambda i: (0, 0)),
            pl.BlockSpec((1, NH), lambda i: (0, 0)),
            pl.BlockSpec((2 * NB, RR), lambda i: (0, 0)),
        ],
        out_specs=pl.BlockSpec((1, 4), lambda i: (0, 0)),
        out_shape=jax.ShapeDtypeStruct((1, 4), jnp.float32),
        scratch_shapes=[
            pltpu.VMEM((2 * HZ + RH, CH), jnp.bfloat16),
            pltpu.VMEM((2 * HZ + RH, CH), jnp.bfloat16),
        ],
        compiler_params=pltpu.CompilerParams(
            dimension_semantics=("arbitrary",)),
    )(ft, fs, aux_flat, W2, bc2, Whead, bhead, pool)

    num_match = jnp.maximum(nm[0, 0], 1.0)
    cls_loss = acc[0, 0] / num_match
    loc_loss = acc[0, 1] / (num_match * 4.0)
    real_loss = acc[0, 2] / num_match
    fake_loss = acc[0, 3] / num_match
    dis_loss = 0.5 * real_loss + 0.5 * fake_loss
    d_loss = 0.6 * loc_loss + 0.6 * cls_loss + 1.8 * dis_loss
    return (d_loss, dis_loss)


# speedup vs baseline: 9.3569x; 9.3569x over previous
import numpy as np
import jax
import jax.numpy as jnp
from jax.experimental import pallas as pl
from jax.experimental.pallas import tpu as pltpu

B = 4
NUM_CLASS = 21
TOP_K = 50
CH = 64
W = 7
NOBJ = 20
OTH = 0.3
CTH = 0.01

PREV = NUM_CLASS * TOP_K
N = B * PREV
P = W * W
SLOTS = 64
OS = 56
NB = 24
NH = 1 + NUM_CLASS + 4
HZ = 16
RH = SLOTS * NB
RR = 2 * RH

_POOL = np.zeros((2 * NB, RR), np.float32)
for _n in range(NB):
    for _y in range(W):
        for _x in range(W):
            _r = _n * SLOTS + 8 * _y + _x
            _POOL[_n, _r] = 1.0 / P
            _POOL[NB + _n, RH + _r] = 1.0 / P


def _match_kernel(prev_ref, tgt_ref, aux_ref, nm_ref):
    b = pl.program_id(0)
    p = prev_ref[0]
    t = tgt_ref[0]
    conf = p[:, 0:1]
    px1, py1, px2, py2 = p[:, 1:2], p[:, 2:3], p[:, 3:4], p[:, 4:5]
    tx1, ty1, tx2, ty2, tcl = t[0:1], t[1:2], t[2:3], t[3:4], t[4:5]

    ix = jnp.clip(jnp.minimum(tx2, px2) - jnp.maximum(tx1, px1), 0.0, None)
    iy = jnp.clip(jnp.minimum(ty2, py2) - jnp.maximum(ty1, py1), 0.0, None)
    inter = ix * iy
    area_t = (tx2 - tx1) * (ty2 - ty1)
    area_p = (px2 - px1) * (py2 - py1)
    ov = inter / (area_t + area_p - inter)

    best = jnp.max(ov, axis=1, keepdims=True)
    li = jax.lax.broadcasted_iota(jnp.int32, ov.shape, 1)
    bidx = jnp.min(jnp.where(ov == best, li, NOBJ), axis=1, keepdims=True)
    sel = li == bidx

    def pick(row):
        return jnp.sum(jnp.where(sel, row, 0.0), axis=1, keepdims=True)

    cls_m = pick(tcl)
    lx1, ly1, lx2, ly2 = pick(tx1), pick(ty1), pick(tx2), pick(ty2)
    m = jnp.logical_and(best >= OTH, conf >= CTH).astype(jnp.float32)

    aux_ref[0] = jnp.concatenate(
        [m, cls_m, lx1, ly1, lx2, ly2, p[:, 1:5],
         jnp.zeros((PREV, 6), jnp.float32)], axis=1)

    @pl.when(b == 0)
    def _():
        nm_ref[0, 0] = 0.0

    nm_ref[0, 0] += jnp.sum(m)


def _disc_kernel(ft_ref, fs_ref, aux_ref, w2_ref, bc_ref, wh_ref, bh_ref,
                 pool_ref, acc_ref, xt_ref, xs_ref, sqt_ref, sqs_ref):
    i = pl.program_id(0)

    @pl.when(i == 0)
    def _():
        xt_ref[0:HZ, :] = jnp.zeros((HZ, CH), jnp.bfloat16)
        xs_ref[0:HZ, :] = jnp.zeros((HZ, CH), jnp.bfloat16)
        xt_ref[HZ + RH:, :] = jnp.zeros((HZ, CH), jnp.bfloat16)
        xs_ref[HZ + RH:, :] = jnp.zeros((HZ, CH), jnp.bfloat16)
        sqt_ref[...] = jnp.zeros((NB, CH, SLOTS), jnp.bfloat16)
        sqs_ref[...] = jnp.zeros((NB, CH, SLOTS), jnp.bfloat16)

    for in_ref, sq_ref, x_ref in ((ft_ref, sqt_ref, xt_ref),
                                  (fs_ref, sqs_ref, xs_ref)):
        xb = in_ref[...].astype(jnp.bfloat16)
        for y in range(W):
            sq_ref[:, :, 8 * y:8 * y + W] = xb[:, :, W * y:W * y + W]
        x_ref[HZ:HZ + RH, :] = \
            jnp.transpose(sq_ref[...], (0, 2, 1)).reshape(RH, CH)

    cols = []
    for x_ref in (xt_ref, xs_ref):
        Xc = x_ref[...]
        cols.append({0: Xc, 1: jnp.roll(Xc, -1, axis=0),
                     -1: jnp.roll(Xc, 1, axis=0)})
    pieces = []
    for ky in range(3):
        for kx in range(3):
            a = HZ + (ky - 1) * 8
            pieces.append(jnp.concatenate(
                [cols[0][kx - 1][a:a + RH, :],
                 cols[1][kx - 1][a:a + RH, :]], axis=0))
    col = jnp.concatenate(pieces, axis=1)

    h = jnp.dot(col, w2_ref[...], preferred_element_type=jnp.float32)
    h = jnp.maximum(h + bc_ref[...], 0.0)

    hb = jnp.dot(pool_ref[...], h.astype(jnp.bfloat16),
                 preferred_element_type=jnp.float32)

    heads = jnp.dot(hb, wh_ref[...], preferred_element_type=jnp.float32)
    heads = heads + bh_ref[...]
    Ht = heads[:NB]
    Hs = heads[NB:]

    aux = aux_ref[...]
    m = aux[:, 0:1]
    cls = aux[:, 1:2].astype(jnp.int32)
    locm = aux[:, 2:6]
    ploc = aux[:, 6:10]

    pt = jnp.clip(jax.nn.sigmoid(Ht[:, 0:1]), 1e-7, 1.0 - 1e-7)
    ps = jnp.clip(jax.nn.sigmoid(Hs[:, 0:1]), 1e-7, 1.0 - 1e-7)
    real = -jnp.log(pt) * m
    fake = -jnp.log(1.0 - ps) * m

    logits = Hs[:, 1:1 + NUM_CLASS]
    mx = jnp.max(logits, axis=1, keepdims=True)
    lz = jnp.log(jnp.sum(jnp.exp(logits - mx), axis=1, keepdims=True)) + mx
    ci = jax.lax.broadcasted_iota(jnp.int32, logits.shape, 1)
    lsel = jnp.sum(jnp.where(ci == cls, logits, 0.0), axis=1, keepdims=True)
    ce = (lz - lsel) * m

    dl = Ht[:, 1 + NUM_CLASS:NH] + ploc
    diff = dl - locm
    ad = jnp.abs(diff)
    sl1 = jnp.where(ad < 1.0, 0.5 * diff * diff, ad - 0.5)
    locc = jnp.sum(sl1, axis=1, keepdims=True) * m

    part = jnp.concatenate([ce, locc, real, fake], axis=1)
    psum = jnp.sum(part, axis=0, keepdims=True)

    @pl.when(i == 0)
    def _():
        acc_ref[...] = jnp.zeros_like(acc_ref)

    acc_ref[...] += psum


def kernel(feature_t, feature_s, prev_t, target, Wc, bc, Wd, bd, Wcl, bcl,
           Wl, bl):
    ft = feature_t.reshape(N, CH, P)
    fs = feature_s.reshape(N, CH, P)
    prev_flat = prev_t.reshape(B, PREV, 5)
    tgt_T = target.transpose(0, 2, 1)
    W2 = Wc.transpose(2, 3, 1, 0).reshape(9 * CH, CH).astype(jnp.bfloat16)
    Whead = jnp.concatenate([Wd, Wcl, Wl], axis=1)
    bhead = jnp.concatenate([bd, bcl, bl], axis=0)[None, :]
    bc2 = bc[None, :]
    pool = jnp.asarray(_POOL, dtype=jnp.bfloat16)

    aux, nm = pl.pallas_call(
        _match_kernel,
        grid=(B,),
        in_specs=[
            pl.BlockSpec((1, PREV, 5), lambda b: (b, 0, 0)),
            pl.BlockSpec((1, 5, NOBJ), lambda b: (b, 0, 0)),
        ],
        out_specs=[
            pl.BlockSpec((1, PREV, 16), lambda b: (b, 0, 0)),
            pl.BlockSpec(memory_space=pltpu.SMEM),
        ],
        out_shape=[
            jax.ShapeDtypeStruct((B, PREV, 16), jnp.float32),
            jax.ShapeDtypeStruct((1, 1), jnp.float32),
        ],
        compiler_params=pltpu.CompilerParams(
            dimension_semantics=("arbitrary",)),
    )(prev_flat, tgt_T)
    aux_flat = aux.reshape(N, 16)

    acc = pl.pallas_call(
        _disc_kernel,
        grid=(N // NB,),
        in_specs=[
            pl.BlockSpec((NB, CH, P), lambda i: (i, 0, 0)),
            pl.BlockSpec((NB, CH, P), lambda i: (i, 0, 0)),
            pl.BlockSpec((NB, 16), lambda i: (i, 0)),
            pl.BlockSpec((9 * CH, CH), lambda i: (0, 0)),
            pl.BlockSpec((1, CH), lambda i: (0, 0)),
            pl.BlockSpec((CH, NH), lambda i: (0, 0)),
            pl.BlockSpec((1, NH), lambda i: (0, 0)),
            pl.BlockSpec((2 * NB, RR), lambda i: (0, 0)),
        ],
        out_specs=pl.BlockSpec((1, 4), lambda i: (0, 0)),
        out_shape=jax.ShapeDtypeStruct((1, 4), jnp.float32),
        scratch_shapes=[
            pltpu.VMEM((2 * HZ + RH, CH), jnp.bfloat16),
            pltpu.VMEM((2 * HZ + RH, CH), jnp.bfloat16),
            pltpu.VMEM((NB, CH, SLOTS), jnp.bfloat16),
            pltpu.VMEM((NB, CH, SLOTS), jnp.bfloat16),
        ],
        compiler_params=pltpu.CompilerParams(
            dimension_semantics=("arbitrary",)),
    )(ft, fs, aux_flat, W2, bc2, Whead, bhead, pool)

    num_match = jnp.maximum(nm[0, 0], 1.0)
    cls_loss = acc[0, 0] / num_match
    loc_loss = acc[0, 1] / (num_match * 4.0)
    real_loss = acc[0, 2] / num_match
    fake_loss = acc[0, 3] / num_match
    dis_loss = 0.5 * real_loss + 0.5 * fake_loss
    d_loss = 0.6 * loc_loss + 0.6 * cls_loss + 1.8 * dis_loss
    return (d_loss, dis_loss)


# speedup vs baseline: 13.6452x; 1.4583x over previous
import numpy as np
import jax
import jax.numpy as jnp
from jax.experimental import pallas as pl
from jax.experimental.pallas import tpu as pltpu

B = 4
NUM_CLASS = 21
TOP_K = 50
CH = 64
W = 7
NOBJ = 20
OTH = 0.3
CTH = 0.01

PREV = NUM_CLASS * TOP_K
N = B * PREV
P = W * W
SLOTS = 64
OS = 56
NB = 56
NH = 1 + NUM_CLASS + 4
HZ = 512
RH = OS * NB
RR = 2 * RH

_POOL = np.zeros((2 * NB, RR), np.float32)
for _n in range(NB):
    for _y in range(W):
        for _x in range(W):
            _r = (8 * _y + _x) * NB + _n
            _POOL[_n, _r] = 1.0 / P
            _POOL[NB + _n, RH + _r] = 1.0 / P


def _match_kernel(prev_ref, tgt_ref, aux_ref, nm_ref):
    b = pl.program_id(0)
    p = prev_ref[0]
    t = tgt_ref[0]
    conf = p[:, 0:1]
    px1, py1, px2, py2 = p[:, 1:2], p[:, 2:3], p[:, 3:4], p[:, 4:5]
    tx1, ty1, tx2, ty2, tcl = t[0:1], t[1:2], t[2:3], t[3:4], t[4:5]

    ix = jnp.clip(jnp.minimum(tx2, px2) - jnp.maximum(tx1, px1), 0.0, None)
    iy = jnp.clip(jnp.minimum(ty2, py2) - jnp.maximum(ty1, py1), 0.0, None)
    inter = ix * iy
    area_t = (tx2 - tx1) * (ty2 - ty1)
    area_p = (px2 - px1) * (py2 - py1)
    ov = inter / (area_t + area_p - inter)

    best = jnp.max(ov, axis=1, keepdims=True)
    li = jax.lax.broadcasted_iota(jnp.int32, ov.shape, 1)
    bidx = jnp.min(jnp.where(ov == best, li, NOBJ), axis=1, keepdims=True)
    sel = li == bidx

    def pick(row):
        return jnp.sum(jnp.where(sel, row, 0.0), axis=1, keepdims=True)

    cls_m = pick(tcl)
    lx1, ly1, lx2, ly2 = pick(tx1), pick(ty1), pick(tx2), pick(ty2)
    m = jnp.logical_and(best >= OTH, conf >= CTH).astype(jnp.float32)

    aux_ref[0] = jnp.concatenate(
        [m, cls_m, lx1, ly1, lx2, ly2, p[:, 1:5],
         jnp.zeros((PREV, 6), jnp.float32)], axis=1)

    @pl.when(b == 0)
    def _():
        nm_ref[0, 0] = 0.0

    nm_ref[0, 0] += jnp.sum(m)


def _disc_kernel(ft_ref, fs_ref, aux_ref, w2_ref, bc_ref, wh_ref, bh_ref,
                 pool_ref, acc_ref, xt_ref, xs_ref):
    i = pl.program_id(0)

    @pl.when(i == 0)
    def _():
        xt_ref[0:HZ, :] = jnp.zeros((HZ, CH), jnp.bfloat16)
        xs_ref[0:HZ, :] = jnp.zeros((HZ, CH), jnp.bfloat16)
        xt_ref[HZ + SLOTS * NB:, :] = jnp.zeros((HZ, CH), jnp.bfloat16)
        xs_ref[HZ + SLOTS * NB:, :] = jnp.zeros((HZ, CH), jnp.bfloat16)

    xt_ref[HZ:HZ + SLOTS * NB, :] = ft_ref[...].reshape(SLOTS * NB, CH)
    xs_ref[HZ:HZ + SLOTS * NB, :] = fs_ref[...].reshape(SLOTS * NB, CH)

    Xt = xt_ref[...]
    Xs = xs_ref[...]
    cols = []
    for ky in range(3):
        for kx in range(3):
            s = (ky - 1) * 8 + (kx - 1)
            a = HZ + s * NB
            cols.append(jnp.concatenate(
                [Xt[a:a + RH, :], Xs[a:a + RH, :]], axis=0))
    col = jnp.concatenate(cols, axis=1)

    h = jnp.dot(col, w2_ref[...], preferred_element_type=jnp.float32)
    h = jnp.maximum(h + bc_ref[...], 0.0)

    hb = jnp.dot(pool_ref[...], h.astype(jnp.bfloat16),
                 preferred_element_type=jnp.float32)

    heads = jnp.dot(hb, wh_ref[...], preferred_element_type=jnp.float32)
    heads = heads + bh_ref[...]
    Ht = heads[:NB]
    Hs = heads[NB:]

    aux = aux_ref[...]
    m = aux[:, 0:1]
    cls = aux[:, 1:2].astype(jnp.int32)
    locm = aux[:, 2:6]
    ploc = aux[:, 6:10]

    pt = jnp.clip(jax.nn.sigmoid(Ht[:, 0:1]), 1e-7, 1.0 - 1e-7)
    ps = jnp.clip(jax.nn.sigmoid(Hs[:, 0:1]), 1e-7, 1.0 - 1e-7)
    real = -jnp.log(pt) * m
    fake = -jnp.log(1.0 - ps) * m

    logits = Hs[:, 1:1 + NUM_CLASS]
    mx = jnp.max(logits, axis=1, keepdims=True)
    lz = jnp.log(jnp.sum(jnp.exp(logits - mx), axis=1, keepdims=True)) + mx
    ci = jax.lax.broadcasted_iota(jnp.int32, logits.shape, 1)
    lsel = jnp.sum(jnp.where(ci == cls, logits, 0.0), axis=1, keepdims=True)
    ce = (lz - lsel) * m

    dl = Ht[:, 1 + NUM_CLASS:NH] + ploc
    diff = dl - locm
    ad = jnp.abs(diff)
    sl1 = jnp.where(ad < 1.0, 0.5 * diff * diff, ad - 0.5)
    locc = jnp.sum(sl1, axis=1, keepdims=True) * m

    part = jnp.concatenate([ce, locc, real, fake], axis=1)
    psum = jnp.sum(part, axis=0, keepdims=True)

    @pl.when(i == 0)
    def _():
        acc_ref[...] = jnp.zeros_like(acc_ref)

    acc_ref[...] += psum


def _pixel_major(f):
    fp = jnp.pad(f.reshape(N, CH, W, W), ((0, 0), (0, 0), (0, 1), (0, 1)))
    fp = fp.reshape(N, CH, SLOTS).astype(jnp.bfloat16)
    fp = jax.lax.optimization_barrier(fp)
    return fp.transpose(2, 0, 1)


def kernel(feature_t, feature_s, prev_t, target, Wc, bc, Wd, bd, Wcl, bcl,
           Wl, bl):
    ft = _pixel_major(feature_t)
    fs = _pixel_major(feature_s)
    prev_flat = prev_t.reshape(B, PREV, 5)
    tgt_T = target.transpose(0, 2, 1)
    W2 = Wc.transpose(2, 3, 1, 0).reshape(9 * CH, CH).astype(jnp.bfloat16)
    Whead = jnp.concatenate([Wd, Wcl, Wl], axis=1)
    bhead = jnp.concatenate([bd, bcl, bl], axis=0)[None, :]
    bc2 = bc[None, :]
    pool = jnp.asarray(_POOL, dtype=jnp.bfloat16)

    aux, nm = pl.pallas_call(
        _match_kernel,
        grid=(B,),
        in_specs=[
            pl.BlockSpec((1, PREV, 5), lambda b: (b, 0, 0)),
            pl.BlockSpec((1, 5, NOBJ), lambda b: (b, 0, 0)),
        ],
        out_specs=[
            pl.BlockSpec((1, PREV, 16), lambda b: (b, 0, 0)),
            pl.BlockSpec(memory_space=pltpu.SMEM),
        ],
        out_shape=[
            jax.ShapeDtypeStruct((B, PREV, 16), jnp.float32),
            jax.ShapeDtypeStruct((1, 1), jnp.float32),
        ],
        compiler_params=pltpu.CompilerParams(
            dimension_semantics=("arbitrary",)),
    )(prev_flat, tgt_T)
    aux_flat = aux.reshape(N, 16)

    acc = pl.pallas_call(
        _disc_kernel,
        grid=(N // NB,),
        in_specs=[
            pl.BlockSpec((SLOTS, NB, CH), lambda i: (0, i, 0)),
            pl.BlockSpec((SLOTS, NB, CH), lambda i: (0, i, 0)),
            pl.BlockSpec((NB, 16), lambda i: (i, 0)),
            pl.BlockSpec((9 * CH, CH), lambda i: (0, 0)),
            pl.BlockSpec((1, CH), lambda i: (0, 0)),
            pl.BlockSpec((CH, NH), lambda i: (0, 0)),
            pl.BlockSpec((1, NH), lambda i: (0, 0)),
            pl.BlockSpec((2 * NB, RR), lambda i: (0, 0)),
        ],
        out_specs=pl.BlockSpec((1, 4), lambda i: (0, 0)),
        out_shape=jax.ShapeDtypeStruct((1, 4), jnp.float32),
        scratch_shapes=[
            pltpu.VMEM((2 * HZ + SLOTS * NB, CH), jnp.bfloat16),
            pltpu.VMEM((2 * HZ + SLOTS * NB, CH), jnp.bfloat16),
        ],
        compiler_params=pltpu.CompilerParams(
            dimension_semantics=("arbitrary",)),
    )(ft, fs, aux_flat, W2, bc2, Whead, bhead, pool)

    num_match = jnp.maximum(nm[0, 0], 1.0)
    cls_loss = acc[0, 0] / num_match
    loc_loss = acc[0, 1] / (num_match * 4.0)
    real_loss = acc[0, 2] / num_match
    fake_loss = acc[0, 3] / num_match
    dis_loss = 0.5 * real_loss + 0.5 * fake_loss
    d_loss = 0.6 * loc_loss + 0.6 * cls_loss + 1.8 * dis_loss
    return (d_loss, dis_loss)
